# baseline (speedup 1.0000x reference)
import functools

import jax
import jax.numpy as jnp
from jax import lax
from jax.experimental import pallas as pl
from jax.experimental.pallas import tpu as pltpu
from jax.experimental.pallas import tpu_sc as plsc

B = 16384
D = 128
V = 1000
NC = 2
NS = 16
L = 16
NW = NC * NS
BPW = B // NW
CH = 64
NCHUNK = BPW // CH


@functools.cache
def _build_sc_scores():
  mesh = plsc.VectorSubcoreMesh(core_axis_name="c", subcore_axis_name="s")

  @functools.partial(
    pl.kernel,
    mesh=mesh,
    out_type=jax.ShapeDtypeStruct((NW, BPW, L), jnp.float32),
    compiler_params=pltpu.CompilerParams(use_tc_tiling_on_sc=False),
    scratch_types=[
        pltpu.VMEM((NCHUNK, CH), jnp.int32),
        pltpu.VMEM((NCHUNK, CH), jnp.int32),
        pltpu.VMEM((NCHUNK, CH), jnp.int32),
        pltpu.VMEM((CH, D), jnp.float32),
        pltpu.VMEM((CH, D), jnp.float32),
        pltpu.VMEM((CH, D), jnp.float32),
        pltpu.VMEM((CH, D), jnp.float32),
        pltpu.VMEM((CH, D), jnp.float32),
        pltpu.VMEM((CH, D), jnp.float32),
        pltpu.VMEM((CH, D), jnp.float32),
        pltpu.VMEM((CH, D), jnp.float32),
        pltpu.VMEM((CH, D), jnp.float32),
        pltpu.VMEM((CH, D), jnp.float32),
        pltpu.VMEM((BPW, L), jnp.float32),
        pltpu.SemaphoreType.DMA,
        pltpu.SemaphoreType.DMA,
    ],
  )
  def _sc_scores(h_hbm, r_hbm, t_hbm, re_ent, im_ent, re_rel, out_hbm,
                 hv, rv, tv,
                 rh0, ih0, rt0, it0, rr0,
                 rh1, ih1, rt1, it1, rr1,
                 outv, sem0, sem1):
    wid = lax.axis_index("s") * NC + lax.axis_index("c")
    bufs = ((rh0, ih0, rt0, it0, rr0), (rh1, ih1, rt1, it1, rr1))
    sems = (sem0, sem1)

    pltpu.sync_copy(h_hbm.at[wid], hv)
    pltpu.sync_copy(r_hbm.at[wid], rv)
    pltpu.sync_copy(t_hbm.at[wid], tv)

    def start(c):
        rh, ih, rt, it, rr = bufs[c % 2]
        sem = sems[c % 2]
        hs, rs, ts = hv.at[c], rv.at[c], tv.at[c]
        return [
            pltpu.async_copy(re_ent.at[hs], rh, sem),
            pltpu.async_copy(im_ent.at[hs], ih, sem),
            pltpu.async_copy(re_ent.at[ts], rt, sem),
            pltpu.async_copy(im_ent.at[ts], it, sem),
            pltpu.async_copy(re_rel.at[rs], rr, sem),
        ]

    def compute(c):
        rh, ih, rt, it, rr = bufs[c % 2]

        def body(cc, carry):
            acc = jnp.zeros((L,), jnp.float32)
            for j in range(D // L):
                sl = pl.ds(j * L, L)
                rhv = rh[cc, sl]
                ihv = ih[cc, sl]
                rtv = rt[cc, sl]
                itv = it[cc, sl]
                rrv = rr[cc, sl]
                acc = acc + rrv * ((rhv - ihv) * rtv + (rhv + ihv) * itv)
            outv[c * CH + cc, :] = acc
            return carry

        lax.fori_loop(0, CH, body, 0, unroll=4)

    pending = start(0)
    for c in range(NCHUNK):
        nxt = start(c + 1) if c + 1 < NCHUNK else None
        for cp in pending:
            cp.wait()
        compute(c)
        pending = nxt

    pltpu.sync_copy(outv, out_hbm.at[wid])

  return _sc_scores


def _bce_body(s_ref, t_ref, o_ref):
    p = s_ref[...]
    grp = (jax.lax.broadcasted_iota(jnp.int32, (D, 8), 0) // L
           == jax.lax.broadcasted_iota(jnp.int32, (D, 8), 1))
    s = jnp.dot(p, grp.astype(jnp.float32),
                preferred_element_type=jnp.float32)
    x = jnp.clip(-s, 0.0, 1.0)
    bce = jnp.maximum(x, 0.0) - x * t_ref[...] + jnp.log1p(jnp.exp(-jnp.abs(x)))
    o_ref[0, 0] = jnp.sum(bce) * (1.0 / B)


_bce_call = pl.pallas_call(
    _bce_body,
    out_shape=jax.ShapeDtypeStruct((1, 1), jnp.float32),
    out_specs=pl.BlockSpec(memory_space=pltpu.SMEM),
)


def kernel(triples, re_ent, im_ent, re_rel, im_rel):
    del im_rel
    h = triples[:, 0].reshape(NW, NCHUNK, CH)
    r = triples[:, 1].reshape(NW, NCHUNK, CH)
    t = triples[:, 2].reshape(NW, NCHUNK, CH)
    tgt = triples[:, 3].astype(jnp.float32).reshape(B // 8, 8)
    partials = _build_sc_scores()(h, r, t, re_ent[:V], im_ent[:V], re_rel[:V])
    loss = _bce_call(partials.reshape(B // 8, D), tgt)
    return loss[0, 0]


# speedup vs baseline: 1.0088x; 1.0088x over previous
import functools

import jax
import jax.numpy as jnp
from jax import lax
from jax.experimental import pallas as pl
from jax.experimental.pallas import tpu as pltpu
from jax.experimental.pallas import tpu_sc as plsc

B = 16384
D = 128
V = 1000
NC = 2
NS = 16
L = 16
NW = NC * NS
BPW = B // NW
CH = 64
NCHUNK = BPW // CH


@functools.cache
def _build_sc_scores():
  mesh = plsc.VectorSubcoreMesh(core_axis_name="c", subcore_axis_name="s")

  @functools.partial(
    pl.kernel,
    mesh=mesh,
    out_type=jax.ShapeDtypeStruct((NW, BPW, L), jnp.float32),
    compiler_params=pltpu.CompilerParams(use_tc_tiling_on_sc=False),
    scratch_types=[
        pltpu.VMEM((NCHUNK, CH), jnp.int32),
        pltpu.VMEM((NCHUNK, CH), jnp.int32),
        pltpu.VMEM((NCHUNK, CH), jnp.int32),
        pltpu.VMEM((CH, D), jnp.float32),
        pltpu.VMEM((CH, D), jnp.float32),
        pltpu.VMEM((CH, D), jnp.float32),
        pltpu.VMEM((CH, D), jnp.float32),
        pltpu.VMEM((CH, D), jnp.float32),
        pltpu.VMEM((CH, D), jnp.float32),
        pltpu.VMEM((CH, D), jnp.float32),
        pltpu.VMEM((CH, D), jnp.float32),
        pltpu.VMEM((CH, D), jnp.float32),
        pltpu.VMEM((CH, D), jnp.float32),
        pltpu.VMEM((BPW, L), jnp.float32),
        pltpu.SemaphoreType.DMA,
        pltpu.SemaphoreType.DMA,
    ],
  )
  def _sc_scores(h_hbm, r_hbm, t_hbm, re_ent, im_ent, re_rel, out_hbm,
                 hv, rv, tv,
                 rh0, ih0, rt0, it0, rr0,
                 rh1, ih1, rt1, it1, rr1,
                 outv, sem0, sem1):
    wid = lax.axis_index("s") * NC + lax.axis_index("c")
    bufs = ((rh0, ih0, rt0, it0, rr0), (rh1, ih1, rt1, it1, rr1))
    sems = (sem0, sem1)

    pltpu.sync_copy(h_hbm.at[wid], hv)
    pltpu.sync_copy(r_hbm.at[wid], rv)
    pltpu.sync_copy(t_hbm.at[wid], tv)

    def start(c):
        rh, ih, rt, it, rr = bufs[c % 2]
        sem = sems[c % 2]
        hs, rs, ts = hv.at[c], rv.at[c], tv.at[c]
        return [
            pltpu.async_copy(re_ent.at[hs], rh, sem),
            pltpu.async_copy(im_ent.at[hs], ih, sem),
            pltpu.async_copy(re_ent.at[ts], rt, sem),
            pltpu.async_copy(im_ent.at[ts], it, sem),
            pltpu.async_copy(re_rel.at[rs], rr, sem),
        ]

    def compute(c):
        rh, ih, rt, it, rr = bufs[c % 2]

        def body(cc, carry):
            acc = jnp.zeros((L,), jnp.float32)
            for j in range(D // L):
                sl = pl.ds(j * L, L)
                rhv = rh[cc, sl]
                ihv = ih[cc, sl]
                rtv = rt[cc, sl]
                itv = it[cc, sl]
                rrv = rr[cc, sl]
                acc = acc + rrv * ((rhv - ihv) * rtv + (rhv + ihv) * itv)
            outv[c * CH + cc, :] = acc
            return carry

        lax.fori_loop(0, CH, body, 0)

    pending = start(0)
    for c in range(NCHUNK):
        nxt = start(c + 1) if c + 1 < NCHUNK else None
        for cp in pending:
            cp.wait()
        compute(c)
        pending = nxt

    pltpu.sync_copy(outv, out_hbm.at[wid])

  return _sc_scores


def _bce_body(s_ref, t_ref, o_ref):
    p = s_ref[...]
    grp = (jax.lax.broadcasted_iota(jnp.int32, (D, 8), 0) // L
           == jax.lax.broadcasted_iota(jnp.int32, (D, 8), 1))
    s = jnp.dot(p, grp.astype(jnp.float32),
                preferred_element_type=jnp.float32)
    x = jnp.clip(-s, 0.0, 1.0)
    bce = jnp.maximum(x, 0.0) - x * t_ref[...] + jnp.log1p(jnp.exp(-jnp.abs(x)))
    o_ref[0, 0] = jnp.sum(bce) * (1.0 / B)


_bce_call = pl.pallas_call(
    _bce_body,
    out_shape=jax.ShapeDtypeStruct((1, 1), jnp.float32),
    out_specs=pl.BlockSpec(memory_space=pltpu.SMEM),
)


def kernel(triples, re_ent, im_ent, re_rel, im_rel):
    del im_rel
    h = triples[:, 0].reshape(NW, NCHUNK, CH)
    r = triples[:, 1].reshape(NW, NCHUNK, CH)
    t = triples[:, 2].reshape(NW, NCHUNK, CH)
    tgt = triples[:, 3].astype(jnp.float32).reshape(B // 8, 8)
    partials = _build_sc_scores()(h, r, t, re_ent[:V], im_ent[:V], re_rel[:V])
    loss = _bce_call(partials.reshape(B // 8, D), tgt)
    return loss[0, 0]


# speedup vs baseline: 1.0415x; 1.0324x over previous
import functools

import jax
import jax.numpy as jnp
from jax import lax
from jax.experimental import pallas as pl
from jax.experimental.pallas import tpu as pltpu
from jax.experimental.pallas import tpu_sc as plsc

B = 16384
D = 128
V = 1000
NC = 2
NS = 16
L = 16
NW = NC * NS
BPW = B // NW
CH = 64
NCHUNK = BPW // CH


@functools.cache
def _build_sc_scores():
  mesh = plsc.VectorSubcoreMesh(core_axis_name="c", subcore_axis_name="s")

  @functools.partial(
    pl.kernel,
    mesh=mesh,
    out_type=jax.ShapeDtypeStruct((NW, BPW, L), jnp.float32),
    compiler_params=pltpu.CompilerParams(use_tc_tiling_on_sc=False),
    scratch_types=[
        pltpu.VMEM((3, NCHUNK, CH), jnp.int32),
        pltpu.VMEM((CH, 2 * D), jnp.float32),
        pltpu.VMEM((CH, 2 * D), jnp.float32),
        pltpu.VMEM((CH, D), jnp.float32),
        pltpu.VMEM((CH, 2 * D), jnp.float32),
        pltpu.VMEM((CH, 2 * D), jnp.float32),
        pltpu.VMEM((CH, D), jnp.float32),
        pltpu.VMEM((BPW, L), jnp.float32),
        pltpu.SemaphoreType.DMA,
        pltpu.SemaphoreType.DMA,
    ],
  )
  def _sc_scores(idx_hbm, ent, rel, out_hbm,
                 iv,
                 he0, te0, rr0,
                 he1, te1, rr1,
                 outv, sem0, sem1):
    wid = lax.axis_index("s") * NC + lax.axis_index("c")
    bufs = ((he0, te0, rr0), (he1, te1, rr1))
    sems = (sem0, sem1)

    pltpu.sync_copy(idx_hbm.at[wid], iv)

    def start(c):
        he, te, rr = bufs[c % 2]
        sem = sems[c % 2]
        hs, rs, ts = iv.at[0, c], iv.at[1, c], iv.at[2, c]
        return [
            pltpu.async_copy(ent.at[hs], he, sem),
            pltpu.async_copy(ent.at[ts], te, sem),
            pltpu.async_copy(rel.at[rs], rr, sem),
        ]

    def compute(c):
        he, te, rr = bufs[c % 2]

        def body(cc, carry):
            acc = jnp.zeros((L,), jnp.float32)
            for j in range(D // L):
                rhv = he[cc, pl.ds(j * L, L)]
                ihv = he[cc, pl.ds(D + j * L, L)]
                rtv = te[cc, pl.ds(j * L, L)]
                itv = te[cc, pl.ds(D + j * L, L)]
                rrv = rr[cc, pl.ds(j * L, L)]
                acc = acc + rrv * ((rhv - ihv) * rtv + (rhv + ihv) * itv)
            outv[c * CH + cc, :] = acc
            return carry

        lax.fori_loop(0, CH, body, 0)

    pending = start(0)
    for c in range(NCHUNK):
        nxt = start(c + 1) if c + 1 < NCHUNK else None
        for cp in pending:
            cp.wait()
        compute(c)
        pending = nxt

    pltpu.sync_copy(outv, out_hbm.at[wid])

  return _sc_scores


def _bce_body(s_ref, t_ref, o_ref):
    p = s_ref[...]
    grp = (jax.lax.broadcasted_iota(jnp.int32, (D, 8), 0) // L
           == jax.lax.broadcasted_iota(jnp.int32, (D, 8), 1))
    s = jnp.dot(p, grp.astype(jnp.float32),
                preferred_element_type=jnp.float32)
    x = jnp.clip(-s, 0.0, 1.0)
    bce = jnp.maximum(x, 0.0) - x * t_ref[...] + jnp.log1p(jnp.exp(-jnp.abs(x)))
    o_ref[0, 0] = jnp.sum(bce) * (1.0 / B)


_bce_call = pl.pallas_call(
    _bce_body,
    out_shape=jax.ShapeDtypeStruct((1, 1), jnp.float32),
    out_specs=pl.BlockSpec(memory_space=pltpu.SMEM),
)


def kernel(triples, re_ent, im_ent, re_rel, im_rel):
    del im_rel
    idx = triples[:, :3].T.reshape(3, NW, NCHUNK, CH).transpose(1, 0, 2, 3)
    tgt = triples[:, 3].astype(jnp.float32).reshape(B // 8, 8)
    ent = jnp.concatenate([re_ent[:V], im_ent[:V]], axis=1)
    partials = _build_sc_scores()(idx, ent, re_rel[:V])
    loss = _bce_call(partials.reshape(B // 8, D), tgt)
    return loss[0, 0]


# speedup vs baseline: 1.0996x; 1.0558x over previous
import functools

import jax
import jax.numpy as jnp
from jax import lax
from jax.experimental import pallas as pl
from jax.experimental.pallas import tpu as pltpu
from jax.experimental.pallas import tpu_sc as plsc

B = 16384
D = 128
V = 1000
NC = 2
NS = 16
L = 16
NW = NC * NS
BPW = B // NW
CH = 64
NCHUNK = BPW // CH


@functools.cache
def _build_sc_scores():
  mesh = plsc.VectorSubcoreMesh(core_axis_name="c", subcore_axis_name="s")

  @functools.partial(
    pl.kernel,
    mesh=mesh,
    out_type=jax.ShapeDtypeStruct((NW, BPW, L), jnp.float32),
    compiler_params=pltpu.CompilerParams(use_tc_tiling_on_sc=False),
    scratch_types=[
        pltpu.VMEM((3, NCHUNK, CH), jnp.int32),
        pltpu.VMEM((CH, 2 * D), jnp.float32),
        pltpu.VMEM((CH, 2 * D), jnp.float32),
        pltpu.VMEM((CH, D), jnp.float32),
        pltpu.VMEM((CH, 2 * D), jnp.float32),
        pltpu.VMEM((CH, 2 * D), jnp.float32),
        pltpu.VMEM((CH, D), jnp.float32),
        pltpu.VMEM((BPW, L), jnp.float32),
        pltpu.SemaphoreType.DMA,
        pltpu.SemaphoreType.DMA,
    ],
  )
  def _sc_scores(idx_hbm, ent, rel, out_hbm,
                 iv,
                 he0, te0, rr0,
                 he1, te1, rr1,
                 outv, sem0, sem1):
    wid = lax.axis_index("s") * NC + lax.axis_index("c")
    bufs = ((he0, te0, rr0), (he1, te1, rr1))
    sems = (sem0, sem1)

    pltpu.sync_copy(idx_hbm.at[wid], iv)

    def start(c):
        he, te, rr = bufs[c % 2]
        sem = sems[c % 2]
        hs, rs, ts = iv.at[0, c], iv.at[1, c], iv.at[2, c]
        return [
            pltpu.async_copy(ent.at[hs], he, sem),
            pltpu.async_copy(ent.at[ts], te, sem),
            pltpu.async_copy(rel.at[rs], rr, sem),
        ]

    def compute(c):
        he, te, rr = bufs[c % 2]

        def body(cc, carry):
            acc = jnp.zeros((L,), jnp.float32)
            for j in range(D // L):
                rhv = he[cc, pl.ds(j * L, L)]
                ihv = he[cc, pl.ds(D + j * L, L)]
                rtv = te[cc, pl.ds(j * L, L)]
                itv = te[cc, pl.ds(D + j * L, L)]
                rrv = rr[cc, pl.ds(j * L, L)]
                acc = acc + rrv * ((rhv - ihv) * rtv + (rhv + ihv) * itv)
            outv[c * CH + cc, :] = acc
            return carry

        lax.fori_loop(0, CH, body, 0)

    pending = start(0)
    for c in range(NCHUNK):
        nxt = start(c + 1) if c + 1 < NCHUNK else None
        for cp in pending:
            cp.wait()
        compute(c)
        pending = nxt

    pltpu.sync_copy(outv, out_hbm.at[wid])

  return _sc_scores


def _bce_body(s_ref, t_ref, o_ref):
    p = s_ref[...]
    grp = (jax.lax.broadcasted_iota(jnp.int32, (D, 8), 0) // L
           == jax.lax.broadcasted_iota(jnp.int32, (D, 8), 1))
    s = jnp.dot(p, grp.astype(jnp.float32),
                preferred_element_type=jnp.float32)
    x = jnp.clip(-s, 0.0, 1.0)
    bce = jnp.maximum(x, 0.0) - x * t_ref[...] + jnp.log1p(jnp.exp(-jnp.abs(x)))
    o_ref[0, 0] = jnp.sum(bce) * (1.0 / B)


_bce_call = pl.pallas_call(
    _bce_body,
    out_shape=jax.ShapeDtypeStruct((1, 1), jnp.float32),
    out_specs=pl.BlockSpec(memory_space=pltpu.SMEM),
)


def kernel(triples, re_ent, im_ent, re_rel, im_rel):
    del im_rel
    idx = triples[:, :3].T.reshape(3, NW, NCHUNK, CH).transpose(1, 0, 2, 3)
    tgt = triples[:, 3].astype(jnp.float32).reshape(B // 8, 8)
    ent = jnp.concatenate([re_ent[:V], im_ent[:V]], axis=1)
    partials = _build_sc_scores()(idx, ent, re_rel)
    loss = _bce_call(partials.reshape(B // 8, D), tgt)
    return loss[0, 0]


# speedup vs baseline: 1.1372x; 1.0342x over previous
import functools

import jax
import jax.numpy as jnp
from jax import lax
from jax.experimental import pallas as pl
from jax.experimental.pallas import tpu as pltpu
from jax.experimental.pallas import tpu_sc as plsc

B = 16384
D = 128
V = 1000
NC = 2
NS = 16
L = 16
NW = NC * NS
BPW = B // NW
CH = 64
NCHUNK = BPW // CH


@functools.cache
def _build_sc_scores():
  mesh = plsc.VectorSubcoreMesh(core_axis_name="c", subcore_axis_name="s")

  @functools.partial(
    pl.kernel,
    mesh=mesh,
    out_type=jax.ShapeDtypeStruct((NW, BPW, L), jnp.float32),
    compiler_params=pltpu.CompilerParams(use_tc_tiling_on_sc=False),
    scratch_types=[
        pltpu.VMEM((3, NCHUNK, CH), jnp.int32),
        pltpu.VMEM((CH, D), jnp.float32),
        pltpu.VMEM((CH, D), jnp.float32),
        pltpu.VMEM((CH, D), jnp.float32),
        pltpu.VMEM((CH, D), jnp.float32),
        pltpu.VMEM((CH, D), jnp.float32),
        pltpu.VMEM((CH, D), jnp.float32),
        pltpu.VMEM((CH, D), jnp.float32),
        pltpu.VMEM((CH, D), jnp.float32),
        pltpu.VMEM((CH, D), jnp.float32),
        pltpu.VMEM((CH, D), jnp.float32),
        pltpu.VMEM((BPW, L), jnp.float32),
        pltpu.SemaphoreType.DMA,
        pltpu.SemaphoreType.DMA,
    ],
  )
  def _sc_scores(idx_hbm, re_ent, im_ent, rel, out_hbm,
                 iv,
                 rh0, ih0, rt0, it0, rr0,
                 rh1, ih1, rt1, it1, rr1,
                 outv, sem0, sem1):
    wid = lax.axis_index("s") * NC + lax.axis_index("c")
    bufs = ((rh0, ih0, rt0, it0, rr0), (rh1, ih1, rt1, it1, rr1))
    sems = (sem0, sem1)

    pltpu.sync_copy(idx_hbm.at[wid], iv)

    def start(c):
        rh, ih, rt, it, rr = bufs[c % 2]
        sem = sems[c % 2]
        hs, rs, ts = iv.at[0, c], iv.at[1, c], iv.at[2, c]
        return [
            pltpu.async_copy(re_ent.at[hs], rh, sem),
            pltpu.async_copy(im_ent.at[hs], ih, sem),
            pltpu.async_copy(re_ent.at[ts], rt, sem),
            pltpu.async_copy(im_ent.at[ts], it, sem),
            pltpu.async_copy(rel.at[rs], rr, sem),
        ]

    def compute(c):
        rh, ih, rt, it, rr = bufs[c % 2]

        def body(cc, carry):
            acc = jnp.zeros((L,), jnp.float32)
            for j in range(D // L):
                sl = pl.ds(j * L, L)
                rhv = rh[cc, sl]
                ihv = ih[cc, sl]
                rtv = rt[cc, sl]
                itv = it[cc, sl]
                rrv = rr[cc, sl]
                acc = acc + rrv * ((rhv - ihv) * rtv + (rhv + ihv) * itv)
            outv[c * CH + cc, :] = acc
            return carry

        lax.fori_loop(0, CH, body, 0)

    pending = start(0)
    for c in range(NCHUNK):
        nxt = start(c + 1) if c + 1 < NCHUNK else None
        for cp in pending:
            cp.wait()
        compute(c)
        pending = nxt

    pltpu.sync_copy(outv, out_hbm.at[wid])

  return _sc_scores


def _bce_body(s_ref, t_ref, o_ref):
    p = s_ref[...]
    grp = (jax.lax.broadcasted_iota(jnp.int32, (D, 8), 0) // L
           == jax.lax.broadcasted_iota(jnp.int32, (D, 8), 1))
    s = jnp.dot(p, grp.astype(jnp.float32),
                preferred_element_type=jnp.float32)
    x = jnp.clip(-s, 0.0, 1.0)
    bce = jnp.maximum(x, 0.0) - x * t_ref[...] + jnp.log1p(jnp.exp(-jnp.abs(x)))
    o_ref[0, 0] = jnp.sum(bce) * (1.0 / B)


_bce_call = pl.pallas_call(
    _bce_body,
    out_shape=jax.ShapeDtypeStruct((1, 1), jnp.float32),
    out_specs=pl.BlockSpec(memory_space=pltpu.SMEM),
)


def kernel(triples, re_ent, im_ent, re_rel, im_rel):
    del im_rel
    idx = triples[:, :3].T.reshape(3, NW, NCHUNK, CH).transpose(1, 0, 2, 3)
    tgt = triples[:, 3].astype(jnp.float32).reshape(B // 8, 8)
    partials = _build_sc_scores()(idx, re_ent, im_ent, re_rel)
    loss = _bce_call(partials.reshape(B // 8, D), tgt)
    return loss[0, 0]
